# baseline (speedup 1.0000x reference)
import functools

import jax
import jax.numpy as jnp
from jax import lax
from jax.experimental import pallas as pl
from jax.experimental.pallas import tpu as pltpu
from jax.experimental.pallas import tpu_sc as plsc

NUM_EMB = 1000000
EMB_DIM = 32
INPUT_LEN = 26
HIDDEN = 256
OUT = 1
B = 16384
IN_DIM = INPUT_LEN * EMB_DIM

N_IDX = B * INPUT_LEN
NUM_WORKERS = 32
N_PER_W = N_IDX // NUM_WORKERS
CHUNK = 512
N_CHUNKS = N_PER_W // CHUNK

PACK = 128 // EMB_DIM
BI = 128
N_BLOCKS = B // (BI * PACK)
HID4 = PACK * HIDDEN


def _sc_gather(x_flat, table):
    mesh = plsc.VectorSubcoreMesh(core_axis_name="c", subcore_axis_name="s")

    @functools.partial(
        pl.kernel,
        mesh=mesh,
        compiler_params=pltpu.CompilerParams(use_tc_tiling_on_sc=False),
        out_type=jax.ShapeDtypeStruct((N_IDX, EMB_DIM), jnp.float32),
        scratch_types=[
            pltpu.VMEM((CHUNK,), jnp.int32),
            pltpu.VMEM((CHUNK, EMB_DIM), jnp.float32),
            pltpu.SemaphoreType.DMA,
        ],
    )
    def gather_kernel(x_hbm, table_hbm, out_hbm, idx_v, buf_v, sem):
        wid = lax.axis_index("s") * 2 + lax.axis_index("c")
        base = wid * N_PER_W

        @pl.loop(0, N_CHUNKS)
        def chunk_loop(c):
            off = base + c * CHUNK
            pltpu.sync_copy(x_hbm.at[pl.ds(off, CHUNK)], idx_v)
            pltpu.async_copy(table_hbm.at[idx_v], buf_v, sem).wait()
            pltpu.sync_copy(buf_v, out_hbm.at[pl.ds(off, CHUNK)])

    return gather_kernel(x_flat, table)


def _mlp_kernel(*refs):
    e4_refs = refs[:INPUT_LEN]
    w1e_ref, b1_ref, w2e_ref, b2_ref, out_ref = refs[INPUT_LEN:]
    h = jnp.dot(
        e4_refs[0][...], w1e_ref[0], preferred_element_type=jnp.float32
    )
    for j in range(1, INPUT_LEN):
        h += jnp.dot(
            e4_refs[j][...], w1e_ref[j], preferred_element_type=jnp.float32
        )
    h = jnp.maximum(h + b1_ref[...], 0.0)
    out_ref[...] = (
        jnp.dot(h, w2e_ref[...], preferred_element_type=jnp.float32)
        + b2_ref[0, 0]
    )


def _e4_spec(j):
    return pl.BlockSpec((BI, 128), lambda i, j=j: (j * N_BLOCKS + i, 0))


def _tc_mlp(e4, W1e, b1_4, W2e, b2):
    return pl.pallas_call(
        _mlp_kernel,
        grid=(N_BLOCKS,),
        in_specs=[_e4_spec(j) for j in range(INPUT_LEN)]
        + [
            pl.BlockSpec((INPUT_LEN, 128, HID4), lambda i: (0, 0, 0)),
            pl.BlockSpec((1, HID4), lambda i: (0, 0)),
            pl.BlockSpec((HID4, PACK), lambda i: (0, 0)),
            pl.BlockSpec((1, 1), lambda i: (0, 0)),
        ],
        out_specs=pl.BlockSpec((BI, PACK), lambda i: (i, 0)),
        out_shape=jax.ShapeDtypeStruct((B // PACK, PACK), jnp.float32),
    )(*([e4] * INPUT_LEN), W1e, b1_4, W2e, b2)


def kernel(x, table, W1, b1, W2, b2):
    table_lin = lax.optimization_barrier(table.reshape(-1))
    table_rm = table_lin.reshape(NUM_EMB, EMB_DIM)

    x_flat = x.T.reshape(-1)
    rows = _sc_gather(x_flat, table_rm)

    e4 = rows.reshape(N_IDX // PACK, 128)

    eye = jnp.eye(PACK, dtype=jnp.float32)
    W1r = W1.reshape(INPUT_LEN, EMB_DIM, HIDDEN)
    W1e = jnp.einsum("jth,qp->jqtph", W1r, eye).reshape(
        INPUT_LEN, 128, HID4
    )
    W2e = jnp.einsum("h,qp->qhp", W2[:, 0], eye).reshape(HID4, PACK)
    b1_4 = jnp.tile(b1, PACK).reshape(1, HID4)
    b2s = b2.reshape(1, 1)

    out4 = _tc_mlp(e4, W1e, b1_4, W2e, b2s)
    return out4.reshape(B, OUT)


# speedup vs baseline: 1.0007x; 1.0007x over previous
import functools

import jax
import jax.numpy as jnp
from jax import lax
from jax.experimental import pallas as pl
from jax.experimental.pallas import tpu as pltpu
from jax.experimental.pallas import tpu_sc as plsc

NUM_EMB = 1000000
EMB_DIM = 32
INPUT_LEN = 26
HIDDEN = 256
OUT = 1
B = 16384
IN_DIM = INPUT_LEN * EMB_DIM

N_IDX = B * INPUT_LEN
NUM_WORKERS = 32
N_PER_W = N_IDX // NUM_WORKERS
CHUNK = 512
N_CHUNKS = N_PER_W // CHUNK

PACK = 128 // EMB_DIM
BI = 128
N_BLOCKS = B // (BI * PACK)
HID4 = PACK * HIDDEN


def _sc_gather(x_flat, table):
    mesh = plsc.VectorSubcoreMesh(core_axis_name="c", subcore_axis_name="s")

    @functools.partial(
        pl.kernel,
        mesh=mesh,
        compiler_params=pltpu.CompilerParams(use_tc_tiling_on_sc=False),
        out_type=jax.ShapeDtypeStruct((N_IDX, EMB_DIM), jnp.float32),
        scratch_types=[
            pltpu.VMEM((CHUNK,), jnp.int32),
            pltpu.VMEM((CHUNK, EMB_DIM), jnp.float32),
            pltpu.SemaphoreType.DMA,
        ],
    )
    def gather_kernel(x_hbm, table_hbm, out_hbm, idx_v, buf_v, sem):
        wid = lax.axis_index("s") * 2 + lax.axis_index("c")
        base = wid * N_PER_W

        @pl.loop(0, N_CHUNKS)
        def chunk_loop(c):
            off = base + c * CHUNK
            pltpu.sync_copy(x_hbm.at[pl.ds(off, CHUNK)], idx_v)
            pltpu.async_copy(table_hbm.at[idx_v], buf_v, sem).wait()
            pltpu.sync_copy(buf_v, out_hbm.at[pl.ds(off, CHUNK)])

    return gather_kernel(x_flat, table)


def _mlp_kernel(*refs):
    e4_refs = refs[:INPUT_LEN]
    w1e_ref, b1_ref, w2e_ref, b2_ref, out_ref = refs[INPUT_LEN:]
    h = jnp.dot(
        e4_refs[0][...], w1e_ref[0], preferred_element_type=jnp.float32
    )
    for j in range(1, INPUT_LEN):
        h += jnp.dot(
            e4_refs[j][...], w1e_ref[j], preferred_element_type=jnp.float32
        )
    h = jnp.maximum(h + b1_ref[...], 0.0)
    out_ref[...] = (
        jnp.dot(h, w2e_ref[...], preferred_element_type=jnp.float32)
        + b2_ref[0, 0]
    )


def _e4_spec(j):
    return pl.BlockSpec((BI, 128), lambda i, j=j: (j * N_BLOCKS + i, 0))


def _tc_mlp(e4, W1e, b1_4, W2e, b2):
    return pl.pallas_call(
        _mlp_kernel,
        grid=(N_BLOCKS,),
        in_specs=[_e4_spec(j) for j in range(INPUT_LEN)]
        + [
            pl.BlockSpec((INPUT_LEN, 128, HID4), lambda i: (0, 0, 0)),
            pl.BlockSpec((1, HID4), lambda i: (0, 0)),
            pl.BlockSpec((HID4, PACK), lambda i: (0, 0)),
            pl.BlockSpec((1, 1), lambda i: (0, 0)),
        ],
        out_specs=pl.BlockSpec((BI, PACK), lambda i: (i, 0)),
        out_shape=jax.ShapeDtypeStruct((B // PACK, PACK), jnp.float32),
    )(*([e4] * INPUT_LEN), W1e, b1_4, W2e, b2)


def kernel(x, table, W1, b1, W2, b2):
    t4 = lax.optimization_barrier(table.reshape(NUM_EMB // PACK, 128))
    table_rm = t4.reshape(NUM_EMB, EMB_DIM)

    x_flat = x.T.reshape(-1)
    rows = _sc_gather(x_flat, table_rm)

    e4 = rows.reshape(N_IDX // PACK, 128)

    eye = jnp.eye(PACK, dtype=jnp.float32)
    W1r = W1.reshape(INPUT_LEN, EMB_DIM, HIDDEN)
    W1e = jnp.einsum("jth,qp->jqtph", W1r, eye).reshape(
        INPUT_LEN, 128, HID4
    )
    W2e = jnp.einsum("h,qp->qhp", W2[:, 0], eye).reshape(HID4, PACK)
    b1_4 = jnp.tile(b1, PACK).reshape(1, HID4)
    b2s = b2.reshape(1, 1)

    out4 = _tc_mlp(e4, W1e, b1_4, W2e, b2s)
    return out4.reshape(B, OUT)


# speedup vs baseline: 1.4996x; 1.4986x over previous
import functools

import jax
import jax.numpy as jnp
from jax import lax
from jax.experimental import pallas as pl
from jax.experimental.pallas import tpu as pltpu
from jax.experimental.pallas import tpu_sc as plsc

NUM_EMB = 1000000
EMB_DIM = 32
INPUT_LEN = 26
HIDDEN = 256
OUT = 1
B = 16384
IN_DIM = INPUT_LEN * EMB_DIM

N_IDX = B * INPUT_LEN
NUM_WORKERS = 32
N_PER_W = N_IDX // NUM_WORKERS
CHUNK = 512
N_CHUNKS = N_PER_W // CHUNK

PACK = 128 // EMB_DIM
BI = 128
N_BLOCKS = B // (BI * PACK)
HID4 = PACK * HIDDEN


def _sc_gather(x_flat, table):
    mesh = plsc.VectorSubcoreMesh(core_axis_name="c", subcore_axis_name="s")

    @functools.partial(
        pl.kernel,
        mesh=mesh,
        compiler_params=pltpu.CompilerParams(use_tc_tiling_on_sc=False),
        out_type=jax.ShapeDtypeStruct((N_IDX, EMB_DIM), jnp.float32),
        scratch_types=[
            pltpu.VMEM((CHUNK,), jnp.int32),
            pltpu.VMEM((CHUNK, EMB_DIM), jnp.float32),
            pltpu.SemaphoreType.DMA,
        ],
    )
    def gather_kernel(x_hbm, table_hbm, out_hbm, idx_v, buf_v, sem):
        wid = lax.axis_index("s") * 2 + lax.axis_index("c")
        base = wid * N_PER_W

        @pl.loop(0, N_CHUNKS)
        def chunk_loop(c):
            off = base + c * CHUNK
            pltpu.sync_copy(x_hbm.at[pl.ds(off, CHUNK)], idx_v)
            pltpu.async_copy(table_hbm.at[idx_v], buf_v, sem).wait()
            pltpu.sync_copy(buf_v, out_hbm.at[pl.ds(off, CHUNK)])

    return gather_kernel(x_flat, table)


def _mlp_kernel(*refs):
    e4_refs = refs[:INPUT_LEN]
    w1e_ref, b1_ref, w2e_ref, b2_ref, out_ref = refs[INPUT_LEN:]
    h = jnp.dot(
        e4_refs[0][...], w1e_ref[0], preferred_element_type=jnp.float32
    )
    for j in range(1, INPUT_LEN):
        h += jnp.dot(
            e4_refs[j][...], w1e_ref[j], preferred_element_type=jnp.float32
        )
    h = jnp.maximum(h + b1_ref[...], 0.0)
    out_ref[...] = (
        jnp.dot(h, w2e_ref[...], preferred_element_type=jnp.float32)
        + b2_ref[0, 0]
    )


def _e4_spec(j):
    return pl.BlockSpec((BI, 128), lambda i, j=j: (j * N_BLOCKS + i, 0))


def _tc_mlp(e4, W1e, b1_4, W2e, b2):
    return pl.pallas_call(
        _mlp_kernel,
        grid=(N_BLOCKS,),
        in_specs=[_e4_spec(j) for j in range(INPUT_LEN)]
        + [
            pl.BlockSpec((INPUT_LEN, 128, HID4), lambda i: (0, 0, 0)),
            pl.BlockSpec((1, HID4), lambda i: (0, 0)),
            pl.BlockSpec((HID4, PACK), lambda i: (0, 0)),
            pl.BlockSpec((1, 1), lambda i: (0, 0)),
        ],
        out_specs=pl.BlockSpec((BI, PACK), lambda i: (i, 0)),
        out_shape=jax.ShapeDtypeStruct((B // PACK, PACK), jnp.float32),
    )(*([e4] * INPUT_LEN), W1e, b1_4, W2e, b2)


REPACK_ROWS = 2048
REPACK_COLS = REPACK_ROWS * PACK
REPACK_GRID = 122
MAIN_EMB = REPACK_GRID * REPACK_COLS
TAIL_EMB = NUM_EMB - MAIN_EMB
TAIL_PACKED = TAIL_EMB // PACK
N_PACKED = NUM_EMB // PACK


def _repack_kernel(t0, t1, t2, t3, out_ref):
    out_ref[...] = jnp.concatenate(
        [t0[...].T, t1[...].T, t2[...].T, t3[...].T], axis=1
    )


def _q_spec(q):
    return pl.BlockSpec(
        (EMB_DIM, REPACK_ROWS), lambda i, q=q: (0, PACK * i + q)
    )


def _tail_kernel(v_ref, big_ref, out_ref):
    del big_ref
    out_ref[...] = v_ref[...]


def _tc_repack(tableT):
    main = pl.pallas_call(
        _repack_kernel,
        grid=(REPACK_GRID,),
        in_specs=[_q_spec(q) for q in range(PACK)],
        out_specs=pl.BlockSpec((REPACK_ROWS, 128), lambda i: (i, 0)),
        out_shape=jax.ShapeDtypeStruct((N_PACKED, 128), jnp.float32),
    )(*([tableT] * PACK))
    v = tableT[:, MAIN_EMB:].T.reshape(TAIL_PACKED, 128)
    return pl.pallas_call(
        _tail_kernel,
        grid=(TAIL_PACKED // 16,),
        in_specs=[
            pl.BlockSpec((16, 128), lambda i: (i, 0)),
            pl.BlockSpec(memory_space=pl.ANY),
        ],
        out_specs=pl.BlockSpec((16, 128), lambda i: (MAIN_EMB // 64 + i, 0)),
        out_shape=jax.ShapeDtypeStruct((N_PACKED, 128), jnp.float32),
        input_output_aliases={1: 0},
    )(v, main)


def kernel(x, table, W1, b1, W2, b2):
    t4 = _tc_repack(table.T)
    table_rm = t4.reshape(NUM_EMB, EMB_DIM)

    x_flat = x.T.reshape(-1)
    k = x_flat // REPACK_COLS
    rem = x_flat - k * REPACK_COLS
    q = rem // REPACK_ROWS
    r_in = rem - q * REPACK_ROWS
    pidx = PACK * (REPACK_ROWS * k + r_in) + q
    pidx = jnp.where(x_flat >= MAIN_EMB, x_flat, pidx)
    rows = _sc_gather(pidx, table_rm)

    e4 = rows.reshape(N_IDX // PACK, 128)

    eye = jnp.eye(PACK, dtype=jnp.float32)
    W1r = W1.reshape(INPUT_LEN, EMB_DIM, HIDDEN)
    W1e = jnp.einsum("jth,qp->jqtph", W1r, eye).reshape(
        INPUT_LEN, 128, HID4
    )
    W2e = jnp.einsum("h,qp->qhp", W2[:, 0], eye).reshape(HID4, PACK)
    b1_4 = jnp.tile(b1, PACK).reshape(1, HID4)
    b2s = b2.reshape(1, 1)

    out4 = _tc_mlp(e4, W1e, b1_4, W2e, b2s)
    return out4.reshape(B, OUT)


# speedup vs baseline: 1.6535x; 1.1026x over previous
import functools

import jax
import jax.numpy as jnp
from jax import lax
from jax.experimental import pallas as pl
from jax.experimental.pallas import tpu as pltpu
from jax.experimental.pallas import tpu_sc as plsc

NUM_EMB = 1000000
EMB_DIM = 32
INPUT_LEN = 26
HIDDEN = 256
OUT = 1
B = 16384
IN_DIM = INPUT_LEN * EMB_DIM

N_IDX = B * INPUT_LEN
NUM_WORKERS = 32
CHUNK = 512

B_H = B // 2
N_IDX_H = B_H * INPUT_LEN
N_PER_W = N_IDX_H // NUM_WORKERS
N_CHUNKS = N_PER_W // CHUNK

PACK = 128 // EMB_DIM
BI = 128
N_BLOCKS = B_H // (BI * PACK)
HID4 = PACK * HIDDEN


def _sc_gather(x_flat, table):
    mesh = plsc.VectorSubcoreMesh(core_axis_name="c", subcore_axis_name="s")

    @functools.partial(
        pl.kernel,
        mesh=mesh,
        compiler_params=pltpu.CompilerParams(use_tc_tiling_on_sc=False),
        out_type=jax.ShapeDtypeStruct((N_IDX_H, EMB_DIM), jnp.float32),
        scratch_types=[
            pltpu.VMEM((CHUNK,), jnp.int32),
            pltpu.VMEM((CHUNK, EMB_DIM), jnp.float32),
            pltpu.SemaphoreType.DMA,
        ],
    )
    def gather_kernel(x_hbm, table_hbm, out_hbm, idx_v, buf_v, sem):
        wid = lax.axis_index("s") * 2 + lax.axis_index("c")
        base = wid * N_PER_W

        @pl.loop(0, N_CHUNKS)
        def chunk_loop(c):
            off = base + c * CHUNK
            pltpu.sync_copy(x_hbm.at[pl.ds(off, CHUNK)], idx_v)
            pltpu.async_copy(table_hbm.at[idx_v], buf_v, sem).wait()
            pltpu.sync_copy(buf_v, out_hbm.at[pl.ds(off, CHUNK)])

    return gather_kernel(x_flat, table)


def _mlp_kernel(*refs):
    e4_refs = refs[:INPUT_LEN]
    w1e_ref, b1_ref, w2e_ref, b2_ref, out_ref = refs[INPUT_LEN:]
    h = jnp.dot(
        e4_refs[0][...], w1e_ref[0], preferred_element_type=jnp.float32
    )
    for j in range(1, INPUT_LEN):
        h += jnp.dot(
            e4_refs[j][...], w1e_ref[j], preferred_element_type=jnp.float32
        )
    h = jnp.maximum(h + b1_ref[...], 0.0)
    out_ref[...] = (
        jnp.dot(h, w2e_ref[...], preferred_element_type=jnp.float32)
        + b2_ref[0, 0]
    )


def _e4_spec(j):
    return pl.BlockSpec((BI, 128), lambda i, j=j: (j * N_BLOCKS + i, 0))


def _tc_mlp(e4, W1e, b1_4, W2e, b2):
    return pl.pallas_call(
        _mlp_kernel,
        grid=(N_BLOCKS,),
        in_specs=[_e4_spec(j) for j in range(INPUT_LEN)]
        + [
            pl.BlockSpec((INPUT_LEN, 128, HID4), lambda i: (0, 0, 0)),
            pl.BlockSpec((1, HID4), lambda i: (0, 0)),
            pl.BlockSpec((HID4, PACK), lambda i: (0, 0)),
            pl.BlockSpec((1, 1), lambda i: (0, 0)),
        ],
        out_specs=pl.BlockSpec((BI, PACK), lambda i: (i, 0)),
        out_shape=jax.ShapeDtypeStruct((B_H // PACK, PACK), jnp.float32),
    )(*([e4] * INPUT_LEN), W1e, b1_4, W2e, b2)


REPACK_ROWS = 4096
REPACK_COLS = REPACK_ROWS * PACK
REPACK_GRID = 61
MAIN_EMB = REPACK_GRID * REPACK_COLS
TAIL_EMB = NUM_EMB - MAIN_EMB
TAIL_PACKED = TAIL_EMB // PACK
N_PACKED = NUM_EMB // PACK


def _repack_kernel(t0, t1, t2, t3, out_ref):
    out_ref[...] = jnp.concatenate(
        [t0[...].T, t1[...].T, t2[...].T, t3[...].T], axis=1
    )


def _q_spec(q):
    return pl.BlockSpec(
        (EMB_DIM, REPACK_ROWS), lambda i, q=q: (0, PACK * i + q)
    )


def _tail_kernel(v_ref, big_ref, out_ref):
    del big_ref
    out_ref[...] = v_ref[...]


def _tc_repack(tableT):
    main = pl.pallas_call(
        _repack_kernel,
        grid=(REPACK_GRID,),
        in_specs=[_q_spec(q) for q in range(PACK)],
        out_specs=pl.BlockSpec((REPACK_ROWS, 128), lambda i: (i, 0)),
        out_shape=jax.ShapeDtypeStruct((N_PACKED, 128), jnp.float32),
    )(*([tableT] * PACK))
    v = tableT[:, MAIN_EMB:].T.reshape(TAIL_PACKED, 128)
    return pl.pallas_call(
        _tail_kernel,
        grid=(TAIL_PACKED // 16,),
        in_specs=[
            pl.BlockSpec((16, 128), lambda i: (i, 0)),
            pl.BlockSpec(memory_space=pl.ANY),
        ],
        out_specs=pl.BlockSpec((16, 128), lambda i: (MAIN_EMB // 64 + i, 0)),
        out_shape=jax.ShapeDtypeStruct((N_PACKED, 128), jnp.float32),
        input_output_aliases={1: 0},
    )(v, main)


def kernel(x, table, W1, b1, W2, b2):
    t4 = _tc_repack(table.T)
    table_rm = t4.reshape(NUM_EMB, EMB_DIM)

    eye = jnp.eye(PACK, dtype=jnp.float32)
    W1r = W1.reshape(INPUT_LEN, EMB_DIM, HIDDEN)
    W1e = jnp.einsum("jth,qp->jqtph", W1r, eye).reshape(
        INPUT_LEN, 128, HID4
    )
    W2e = jnp.einsum("h,qp->qhp", W2[:, 0], eye).reshape(HID4, PACK)
    b1_4 = jnp.tile(b1, PACK).reshape(1, HID4)
    b2s = b2.reshape(1, 1)

    xT = x.T
    outs = []
    for hh in range(2):
        x_flat = xT[:, hh * B_H:(hh + 1) * B_H].reshape(-1)
        k = x_flat // REPACK_COLS
        rem = x_flat - k * REPACK_COLS
        q = rem // REPACK_ROWS
        r_in = rem - q * REPACK_ROWS
        pidx = PACK * (REPACK_ROWS * k + r_in) + q
        pidx = jnp.where(x_flat >= MAIN_EMB, x_flat, pidx)
        rows = _sc_gather(pidx, table_rm)
        e4 = rows.reshape(N_IDX_H // PACK, 128)
        outs.append(_tc_mlp(e4, W1e, b1_4, W2e, b2s))

    return jnp.concatenate(outs, axis=0).reshape(B, OUT)
